# baseline (speedup 1.0000x reference)
import functools

import jax
import jax.numpy as jnp
from jax.experimental import pallas as pl
from jax.experimental.pallas import tpu as pltpu

B, L, M, D = 4, 2048, 8, 768
NR = 4
TL = 512


def _index_body(noise_ref, pm_ref, rev_ref, rem_ref, msk_ref, rmask_ref,
                bits_ref):
    n = noise_ref[0]

    ranks = []
    for m in range(M):
        nm = n[m:m + 1, :]
        acc = jnp.zeros((1, L), dtype=jnp.int32)
        for mp in range(M):
            if mp == m:
                continue
            nmp = n[mp:mp + 1, :]
            lt = nmp < nm
            if mp < m:
                lt = jnp.logical_or(lt, nmp == nm)
            acc = acc + lt.astype(jnp.int32)
        ranks.append(acc)
        rev_ref[0, m:m + 1, :] = acc

    for r in range(NR):
        rem = jnp.zeros((1, L), dtype=jnp.int32)
        msk = jnp.zeros((1, L), dtype=jnp.int32)
        for m in range(M):
            mi = jnp.int32(m)
            rem = rem + jnp.where(ranks[m] == r, mi, 0)
            msk = msk + jnp.where(ranks[m] == r + NR, mi, 0)
        rem_ref[0, r:r + 1, :] = rem
        msk_ref[0, r:r + 1, :] = msk
        for k in range(3):
            bits_ref[0, r * 3 + k:r * 3 + k + 1, :] = (
                jnp.right_shift(rem, k) & 1)
        rmask_ref[0, r:r + 1, :] = pm_ref[0]


def _data_body(b0r, b1r, b2r, b3r, d0, d1, d2, d3, d4, d5, d6, d7, out_ref):
    data = (d0[0], d1[0], d2[0], d3[0], d4[0], d5[0], d6[0], d7[0])
    bits = (b0r, b1r, b2r, b3r)
    for r in range(NR):
        bb = bits[r][0]
        b0 = bb[:, 0:1] != 0
        b1 = bb[:, 1:2] != 0
        b2 = bb[:, 2:3] != 0
        t0 = jnp.where(b0, data[1], data[0])
        t1 = jnp.where(b0, data[3], data[2])
        t2 = jnp.where(b0, data[5], data[4])
        t3 = jnp.where(b0, data[7], data[6])
        u0 = jnp.where(b1, t1, t0)
        u1 = jnp.where(b1, t3, t2)
        out_ref[0, :, r * D:(r + 1) * D] = jnp.where(b2, u1, u0)


@jax.jit
def _run(noise_t, pm_t, data):
    rev_t, rem_t, msk_t, rmask_t, bits_t = pl.pallas_call(
        _index_body,
        grid=(B,),
        in_specs=[
            pl.BlockSpec((1, M, L), lambda b: (b, 0, 0)),
            pl.BlockSpec((1, 1, L), lambda b: (b, 0, 0)),
        ],
        out_specs=[
            pl.BlockSpec((1, M, L), lambda b: (b, 0, 0)),
            pl.BlockSpec((1, NR, L), lambda b: (b, 0, 0)),
            pl.BlockSpec((1, NR, L), lambda b: (b, 0, 0)),
            pl.BlockSpec((1, NR, L), lambda b: (b, 0, 0)),
            pl.BlockSpec((1, 3 * NR, L), lambda b: (b, 0, 0)),
        ],
        out_shape=[
            jax.ShapeDtypeStruct((B, M, L), jnp.int32),
            jax.ShapeDtypeStruct((B, NR, L), jnp.int32),
            jax.ShapeDtypeStruct((B, NR, L), jnp.int32),
            jax.ShapeDtypeStruct((B, NR, L), jnp.float32),
            jax.ShapeDtypeStruct((B, 3 * NR, L), jnp.int32),
        ],
        compiler_params=pltpu.CompilerParams(
            dimension_semantics=("parallel",),
        ),
    )(noise_t, pm_t)

    bits = [bits_t[:, r * 3:(r + 1) * 3, :].transpose(0, 2, 1) for r in range(NR)]

    data_spec = pl.BlockSpec((1, TL, D), lambda b, i: (b, i, 0))
    bits_spec = pl.BlockSpec((1, TL, 3), lambda b, i: (b, i, 0))
    remained = pl.pallas_call(
        _data_body,
        grid=(B, L // TL),
        in_specs=[bits_spec] * NR + [data_spec] * M,
        out_specs=pl.BlockSpec((1, TL, NR * D), lambda b, i: (b, i, 0)),
        out_shape=jax.ShapeDtypeStruct((B, L, NR * D), jnp.float32),
        compiler_params=pltpu.CompilerParams(
            dimension_semantics=("parallel", "parallel"),
        ),
    )(*bits, *data)
    return remained, rev_t, rem_t, msk_t, rmask_t


def kernel(data_0, data_1, data_2, data_3, data_4, data_5, data_6, data_7,
           temporal_padding_mask):
    data = (data_0, data_1, data_2, data_3, data_4, data_5, data_6, data_7)
    noise_t = jax.random.uniform(jax.random.key(42), (B, L, M)).transpose(0, 2, 1)
    pm = jnp.concatenate(
        [jnp.ones((B, 1, 1), temporal_padding_mask.dtype), temporal_padding_mask],
        axis=1)
    pm_t = pm.transpose(0, 2, 1)
    remained, rev_t, rem_t, msk_t, rmask_t = _run(noise_t, pm_t, data)
    remained_data = remained.reshape(B, L, NR, D)
    remain_idx = rem_t.transpose(0, 2, 1)
    masked_idx = msk_t.transpose(0, 2, 1)
    revert_idx = rev_t.transpose(0, 2, 1)
    remain_padding_mask = rmask_t.transpose(0, 2, 1)
    return (remained_data, remain_padding_mask, remain_idx, masked_idx,
            revert_idx, pm)


# speedup vs baseline: 1.0121x; 1.0121x over previous
import functools
import math

import numpy as np

import jax
import jax.numpy as jnp
from jax import lax
from jax.experimental import pallas as pl
from jax.experimental.pallas import tpu as pltpu
from jax.experimental.pallas import tpu_sc as plsc

B, L, M, D = 4, 2048, 8, 768
NR = 4
BL = B * L
NW = 32
CH = 64


def _build_routing():
    noise = np.asarray(
        jax.random.uniform(jax.random.key(42), (B, L, M))).reshape(BL, M)
    order = np.argsort(noise, axis=-1, kind="stable")
    remain = order[:, :NR]
    per_m = []
    for m in range(M):
        t_idx, r_idx = np.nonzero(remain == m)
        g = t_idx.astype(np.int32)
        s = (NR * t_idx + r_idx).astype(np.int32)
        njob = math.ceil(len(g) / CH)
        pad = njob * CH - len(g)
        if pad:
            last = (njob - 1) * CH
            g = np.concatenate([g, np.full(pad, g[last], np.int32)])
            s = np.concatenate([s, np.full(pad, s[last], np.int32)])
        per_m.append([g.reshape(njob, CH), s.reshape(njob, CH)])
    job_base = [0]
    for g, _ in per_m:
        job_base.append(job_base[-1] + g.shape[0])
    gidx = np.concatenate([g for g, _ in per_m], axis=0)
    sidx = np.concatenate([s for _, s in per_m], axis=0)
    n_jobs = job_base[-1]
    jpw = math.ceil(n_jobs / NW)
    pad_rows = NW * jpw - n_jobs
    if pad_rows:
        gpad = np.full((pad_rows, CH), gidx[0, 0], np.int32)
        spad = np.full((pad_rows, CH), sidx[0, 0], np.int32)
        gidx = np.concatenate([gidx, gpad], axis=0)
        sidx = np.concatenate([sidx, spad], axis=0)
    return (gidx.reshape(NW, jpw, CH), sidx.reshape(NW, jpw, CH),
            tuple(job_base), n_jobs, jpw)


_GIDX_NP, _SIDX_NP, _JOB_BASE, _N_JOBS, _JPW = _build_routing()


_SC_MESH = plsc.VectorSubcoreMesh(core_axis_name="c", subcore_axis_name="s")


@functools.partial(
    pl.kernel,
    out_type=jax.ShapeDtypeStruct((BL * NR, D), jnp.float32),
    mesh=_SC_MESH,
    scratch_types=[
        pltpu.VMEM((_JPW, CH), jnp.int32),
        pltpu.VMEM((_JPW, CH), jnp.int32),
        pltpu.VMEM((CH, D), jnp.float32),
        pltpu.VMEM((CH, D), jnp.float32),
        pltpu.SemaphoreType.DMA,
        pltpu.SemaphoreType.DMA,
        pltpu.SemaphoreType.DMA,
    ],
)
def _sc_gather(d0, d1, d2, d3, d4, d5, d6, d7, gidx_hbm, sidx_hbm, out_hbm,
               gv, sv, buf0, buf1, gsem, ssem0, ssem1):
    data = (d0, d1, d2, d3, d4, d5, d6, d7)
    bufs = (buf0, buf1)
    ssems = (ssem0, ssem1)
    wid = lax.axis_index("c") * 16 + lax.axis_index("s")
    base = wid * _JPW
    pltpu.sync_copy(gidx_hbm.at[wid], gv)
    pltpu.sync_copy(sidx_hbm.at[wid], sv)
    for k in range(_JPW):
        j = base + k
        p = k & 1

        @pl.when(j < _N_JOBS)
        def _process(k=k, p=p, j=j):
            if k >= 2:
                pltpu.make_async_copy(
                    bufs[p], out_hbm.at[sv.at[k - 2]], ssems[p]).wait()
            for m in range(M):
                lo, hi = _JOB_BASE[m], _JOB_BASE[m + 1]
                if hi == lo:
                    continue

                @pl.when(jnp.logical_and(j >= lo, j < hi))
                def _gather(m=m):
                    pltpu.async_copy(data[m].at[gv.at[k]], bufs[p], gsem).wait()

            pltpu.async_copy(bufs[p], out_hbm.at[sv.at[k]], ssems[p])

    for k in range(_JPW):
        p = k & 1
        valid_k = base + k < _N_JOBS
        if k + 2 >= _JPW:
            cond = valid_k
        else:
            cond = jnp.logical_and(valid_k, base + k + 2 >= _N_JOBS)

        @pl.when(cond)
        def _drain(k=k, p=p):
            pltpu.make_async_copy(
                bufs[p], out_hbm.at[sv.at[k]], ssems[p]).wait()


def _index_body(noise_ref, pm_ref, rev_ref, rem_ref, msk_ref, rmask_ref):
    n = noise_ref[0]

    ranks = []
    for m in range(M):
        nm = n[m:m + 1, :]
        acc = jnp.zeros((1, L), dtype=jnp.int32)
        for mp in range(M):
            if mp == m:
                continue
            nmp = n[mp:mp + 1, :]
            lt = nmp < nm
            if mp < m:
                lt = jnp.logical_or(lt, nmp == nm)
            acc = acc + lt.astype(jnp.int32)
        ranks.append(acc)
        rev_ref[0, m:m + 1, :] = acc

    for r in range(NR):
        rem = jnp.zeros((1, L), dtype=jnp.int32)
        msk = jnp.zeros((1, L), dtype=jnp.int32)
        for m in range(M):
            mi = jnp.int32(m)
            rem = rem + jnp.where(ranks[m] == r, mi, 0)
            msk = msk + jnp.where(ranks[m] == r + NR, mi, 0)
        rem_ref[0, r:r + 1, :] = rem
        msk_ref[0, r:r + 1, :] = msk
        rmask_ref[0, r:r + 1, :] = pm_ref[0]


@jax.jit
def _run(noise_t, pm_t, data):
    rev_t, rem_t, msk_t, rmask_t = pl.pallas_call(
        _index_body,
        grid=(B,),
        in_specs=[
            pl.BlockSpec((1, M, L), lambda b: (b, 0, 0)),
            pl.BlockSpec((1, 1, L), lambda b: (b, 0, 0)),
        ],
        out_specs=[
            pl.BlockSpec((1, M, L), lambda b: (b, 0, 0)),
            pl.BlockSpec((1, NR, L), lambda b: (b, 0, 0)),
            pl.BlockSpec((1, NR, L), lambda b: (b, 0, 0)),
            pl.BlockSpec((1, NR, L), lambda b: (b, 0, 0)),
        ],
        out_shape=[
            jax.ShapeDtypeStruct((B, M, L), jnp.int32),
            jax.ShapeDtypeStruct((B, NR, L), jnp.int32),
            jax.ShapeDtypeStruct((B, NR, L), jnp.int32),
            jax.ShapeDtypeStruct((B, NR, L), jnp.float32),
        ],
        compiler_params=pltpu.CompilerParams(
            dimension_semantics=("parallel",),
        ),
    )(noise_t, pm_t)

    flat = [d.reshape(BL, D) for d in data]
    remained = _sc_gather(*flat, jnp.asarray(_GIDX_NP), jnp.asarray(_SIDX_NP))
    return remained, rev_t, rem_t, msk_t, rmask_t


def kernel(data_0, data_1, data_2, data_3, data_4, data_5, data_6, data_7,
           temporal_padding_mask):
    data = (data_0, data_1, data_2, data_3, data_4, data_5, data_6, data_7)
    noise_t = jax.random.uniform(jax.random.key(42), (B, L, M)).transpose(0, 2, 1)
    pm = jnp.concatenate(
        [jnp.ones((B, 1, 1), temporal_padding_mask.dtype), temporal_padding_mask],
        axis=1)
    pm_t = pm.transpose(0, 2, 1)
    remained, rev_t, rem_t, msk_t, rmask_t = _run(noise_t, pm_t, data)
    remained_data = remained.reshape(B, L, NR, D)
    remain_idx = rem_t.transpose(0, 2, 1)
    masked_idx = msk_t.transpose(0, 2, 1)
    revert_idx = rev_t.transpose(0, 2, 1)
    remain_padding_mask = rmask_t.transpose(0, 2, 1)
    return (remained_data, remain_padding_mask, remain_idx, masked_idx,
            revert_idx, pm)


# speedup vs baseline: 1.0260x; 1.0137x over previous
import functools
import math

import numpy as np

import jax
import jax.numpy as jnp
from jax import lax
from jax.experimental import pallas as pl
from jax.experimental.pallas import tpu as pltpu
from jax.experimental.pallas import tpu_sc as plsc

B, L, M, D = 4, 2048, 8, 768
NR = 4
BL = B * L
NW = 32
CH = 73


def _build_routing():
    noise = np.asarray(
        jax.random.uniform(jax.random.key(42), (B, L, M))).reshape(BL, M)
    order = np.argsort(noise, axis=-1, kind="stable")
    remain = order[:, :NR]
    per_m = []
    for m in range(M):
        t_idx, r_idx = np.nonzero(remain == m)
        g = t_idx.astype(np.int32)
        s = (NR * t_idx + r_idx).astype(np.int32)
        njob = math.ceil(len(g) / CH)
        pad = njob * CH - len(g)
        if pad:
            last = (njob - 1) * CH
            g = np.concatenate([g, np.full(pad, g[last], np.int32)])
            s = np.concatenate([s, np.full(pad, s[last], np.int32)])
        per_m.append([g.reshape(njob, CH), s.reshape(njob, CH)])
    job_base = [0]
    for g, _ in per_m:
        job_base.append(job_base[-1] + g.shape[0])
    gidx = np.concatenate([g for g, _ in per_m], axis=0)
    sidx = np.concatenate([s for _, s in per_m], axis=0)
    n_jobs = job_base[-1]
    jpw = math.ceil(n_jobs / NW)
    pad_rows = NW * jpw - n_jobs
    if pad_rows:
        gpad = np.full((pad_rows, CH), gidx[0, 0], np.int32)
        spad = np.full((pad_rows, CH), sidx[0, 0], np.int32)
        gidx = np.concatenate([gidx, gpad], axis=0)
        sidx = np.concatenate([sidx, spad], axis=0)
    return (gidx.reshape(NW, jpw, CH), sidx.reshape(NW, jpw, CH),
            tuple(job_base), n_jobs, jpw)


_GIDX_NP, _SIDX_NP, _JOB_BASE, _N_JOBS, _JPW = _build_routing()


_SC_MESH = plsc.VectorSubcoreMesh(core_axis_name="c", subcore_axis_name="s")


@functools.partial(
    pl.kernel,
    out_type=jax.ShapeDtypeStruct((BL * NR, D), jnp.float32),
    mesh=_SC_MESH,
    scratch_types=[
        pltpu.VMEM((_JPW, CH), jnp.int32),
        pltpu.VMEM((_JPW, CH), jnp.int32),
        pltpu.VMEM((CH, D), jnp.float32),
        pltpu.VMEM((CH, D), jnp.float32),
        pltpu.SemaphoreType.DMA,
        pltpu.SemaphoreType.DMA,
        pltpu.SemaphoreType.DMA,
    ],
)
def _sc_gather(d0, d1, d2, d3, d4, d5, d6, d7, gidx_hbm, sidx_hbm, out_hbm,
               gv, sv, buf0, buf1, gsem, ssem0, ssem1):
    data = (d0, d1, d2, d3, d4, d5, d6, d7)
    bufs = (buf0, buf1)
    ssems = (ssem0, ssem1)
    wid = lax.axis_index("c") * 16 + lax.axis_index("s")
    base = wid * _JPW
    pltpu.sync_copy(gidx_hbm.at[wid], gv)
    pltpu.sync_copy(sidx_hbm.at[wid], sv)
    for k in range(_JPW):
        j = base + k
        p = k & 1

        @pl.when(j < _N_JOBS)
        def _process(k=k, p=p, j=j):
            if k >= 2:
                pltpu.make_async_copy(
                    bufs[p], out_hbm.at[sv.at[k - 2]], ssems[p]).wait()
            for m in range(M):
                lo, hi = _JOB_BASE[m], _JOB_BASE[m + 1]
                if hi == lo:
                    continue

                @pl.when(jnp.logical_and(j >= lo, j < hi))
                def _gather(m=m):
                    pltpu.async_copy(data[m].at[gv.at[k]], bufs[p], gsem).wait()

            pltpu.async_copy(bufs[p], out_hbm.at[sv.at[k]], ssems[p])

    for k in range(_JPW):
        p = k & 1
        valid_k = base + k < _N_JOBS
        if k + 2 >= _JPW:
            cond = valid_k
        else:
            cond = jnp.logical_and(valid_k, base + k + 2 >= _N_JOBS)

        @pl.when(cond)
        def _drain(k=k, p=p):
            pltpu.make_async_copy(
                bufs[p], out_hbm.at[sv.at[k]], ssems[p]).wait()


def _index_body(noise_ref, pm_ref, rev_ref, rem_ref, msk_ref, rmask_ref):
    n = noise_ref[0]

    ranks = []
    for m in range(M):
        nm = n[m:m + 1, :]
        acc = jnp.zeros((1, L), dtype=jnp.int32)
        for mp in range(M):
            if mp == m:
                continue
            nmp = n[mp:mp + 1, :]
            lt = nmp < nm
            if mp < m:
                lt = jnp.logical_or(lt, nmp == nm)
            acc = acc + lt.astype(jnp.int32)
        ranks.append(acc)
        rev_ref[0, m:m + 1, :] = acc

    for r in range(NR):
        rem = jnp.zeros((1, L), dtype=jnp.int32)
        msk = jnp.zeros((1, L), dtype=jnp.int32)
        for m in range(M):
            mi = jnp.int32(m)
            rem = rem + jnp.where(ranks[m] == r, mi, 0)
            msk = msk + jnp.where(ranks[m] == r + NR, mi, 0)
        rem_ref[0, r:r + 1, :] = rem
        msk_ref[0, r:r + 1, :] = msk
        rmask_ref[0, r:r + 1, :] = pm_ref[0]


@jax.jit
def _run(noise_t, pm_t, data):
    rev_t, rem_t, msk_t, rmask_t = pl.pallas_call(
        _index_body,
        grid=(B,),
        in_specs=[
            pl.BlockSpec((1, M, L), lambda b: (b, 0, 0)),
            pl.BlockSpec((1, 1, L), lambda b: (b, 0, 0)),
        ],
        out_specs=[
            pl.BlockSpec((1, M, L), lambda b: (b, 0, 0)),
            pl.BlockSpec((1, NR, L), lambda b: (b, 0, 0)),
            pl.BlockSpec((1, NR, L), lambda b: (b, 0, 0)),
            pl.BlockSpec((1, NR, L), lambda b: (b, 0, 0)),
        ],
        out_shape=[
            jax.ShapeDtypeStruct((B, M, L), jnp.int32),
            jax.ShapeDtypeStruct((B, NR, L), jnp.int32),
            jax.ShapeDtypeStruct((B, NR, L), jnp.int32),
            jax.ShapeDtypeStruct((B, NR, L), jnp.float32),
        ],
        compiler_params=pltpu.CompilerParams(
            dimension_semantics=("parallel",),
        ),
    )(noise_t, pm_t)

    flat = [d.reshape(BL, D) for d in data]
    remained = _sc_gather(*flat, jnp.asarray(_GIDX_NP), jnp.asarray(_SIDX_NP))
    return remained, rev_t, rem_t, msk_t, rmask_t


def kernel(data_0, data_1, data_2, data_3, data_4, data_5, data_6, data_7,
           temporal_padding_mask):
    data = (data_0, data_1, data_2, data_3, data_4, data_5, data_6, data_7)
    noise_t = jax.random.uniform(jax.random.key(42), (B, L, M)).transpose(0, 2, 1)
    pm = jnp.concatenate(
        [jnp.ones((B, 1, 1), temporal_padding_mask.dtype), temporal_padding_mask],
        axis=1)
    pm_t = pm.transpose(0, 2, 1)
    remained, rev_t, rem_t, msk_t, rmask_t = _run(noise_t, pm_t, data)
    remained_data = remained.reshape(B, L, NR, D)
    remain_idx = rem_t.transpose(0, 2, 1)
    masked_idx = msk_t.transpose(0, 2, 1)
    revert_idx = rev_t.transpose(0, 2, 1)
    remain_padding_mask = rmask_t.transpose(0, 2, 1)
    return (remained_data, remain_padding_mask, remain_idx, masked_idx,
            revert_idx, pm)
